# baseline (speedup 1.0000x reference)
import functools

import jax
import jax.numpy as jnp
from jax import lax
from jax.experimental import pallas as pl
from jax.experimental.pallas import tpu as pltpu
from jax.experimental.pallas import tpu_sc as plsc

N = 10000
E = 320000
HID = 128

NC, NS = 2, 16
NW = NC * NS
EPW = E // NW
CHUNK = 125
NCHUNK = EPW // CHUNK
HCHUNK = NCHUNK // 2
NPAD = 10240
RPT = NPAD // NS

RB = 1000
GRID_R = N // RB
GBM = 1024
GBN = 4096
GRID_GM = pl.cdiv(N, GBM)
GRID_GN = pl.cdiv(N, GBN)


def _segsum_body(table, src3, dst3, zeros, out, src_v, dst_v, rows0_v,
                 rows1_v, agg_sh, sem0, sem1):
    c = lax.axis_index("c")
    s = lax.axis_index("s")
    wid = c * NS + s
    pltpu.sync_copy(zeros, agg_sh.at[pl.ds(s * RPT, RPT)])
    plsc.subcore_barrier()

    for half in range(2):
        pltpu.sync_copy(src3.at[wid, pl.ds(half * HCHUNK, HCHUNK)], src_v)
        pltpu.sync_copy(dst3.at[wid, pl.ds(half * HCHUNK, HCHUNK)], dst_v)
        pltpu.async_copy(table.at[src_v.at[0]], rows0_v, sem0)

        def pair(t, carry):
            j0 = 2 * t
            pltpu.make_async_copy(table.at[src_v.at[j0]], rows0_v,
                                  sem0).wait()
            pltpu.async_copy(table.at[src_v.at[j0 + 1]], rows1_v, sem1)
            pltpu.sync_copy(rows0_v, agg_sh.at[dst_v.at[j0]], add=True)
            pltpu.make_async_copy(table.at[src_v.at[j0 + 1]], rows1_v,
                                  sem1).wait()

            @pl.when(t + 1 < HCHUNK // 2)
            def _():
                pltpu.async_copy(table.at[src_v.at[j0 + 2]], rows0_v, sem0)

            pltpu.sync_copy(rows1_v, agg_sh.at[dst_v.at[j0 + 1]], add=True)
            return carry

        lax.fori_loop(0, HCHUNK // 2, pair, 0)
    plsc.subcore_barrier()
    pltpu.sync_copy(agg_sh.at[pl.ds(s * RPT, RPT)],
                    out.at[c, pl.ds(s * RPT, RPT)])


@functools.lru_cache(maxsize=1)
def _build_segsum():
    return pl.kernel(
        _segsum_body,
        out_type=jax.ShapeDtypeStruct((NC, NPAD, HID), jnp.float32),
        mesh=plsc.VectorSubcoreMesh(core_axis_name="c", subcore_axis_name="s",
                                    num_cores=NC, num_subcores=NS),
        scratch_types=[
            pltpu.VMEM((HCHUNK, CHUNK), jnp.int32),
            pltpu.VMEM((HCHUNK, CHUNK), jnp.int32),
            pltpu.VMEM((CHUNK, HID), jnp.float32),
            pltpu.VMEM((CHUNK, HID), jnp.float32),
            pltpu.VMEM_SHARED((NPAD, HID), jnp.float32),
            pltpu.SemaphoreType.DMA,
            pltpu.SemaphoreType.DMA,
        ],
    )


def _segsum(table, src3, dst3, zeros):
    return _build_segsum()(table, src3, dst3, zeros)


def _lin_relu_body(p0_ref, p1_ref, w_ref, b_ref, o_ref):
    s = p0_ref[0] + p1_ref[0]
    o_ref[...] = jnp.maximum(
        jnp.dot(s, w_ref[...], preferred_element_type=jnp.float32)
        + b_ref[...], 0.0)


def _lin_relu(p, w, b):
    return pl.pallas_call(
        _lin_relu_body,
        grid=(GRID_R,),
        in_specs=[
            pl.BlockSpec((1, RB, HID), lambda i: (0, i, 0)),
            pl.BlockSpec((1, RB, HID), lambda i: (1, i, 0)),
            pl.BlockSpec((HID, HID), lambda i: (0, 0)),
            pl.BlockSpec((1, HID), lambda i: (0, 0)),
        ],
        out_specs=pl.BlockSpec((RB, HID), lambda i: (i, 0)),
        out_shape=jax.ShapeDtypeStruct((N, HID), jnp.float32),
    )(p, p, w, b)


def _mix_body(x_ref, q0_ref, q1_ref, wfc_ref, bfc_ref, w2_ref, b2_ref,
              eps_ref, h_ref, stats_ref):
    h1 = (jnp.dot(x_ref[...], wfc_ref[...],
                  preferred_element_type=jnp.float32) + bfc_ref[...])
    agg = q0_ref[0] + q1_ref[0]
    h2 = (jnp.dot(agg, w2_ref[...],
                  preferred_element_type=jnp.float32) + b2_ref[...])
    e = eps_ref[...]
    h = h1 + e * (h2 - h1)
    h_ref[...] = h

    @pl.when(pl.program_id(0) == 0)
    def _():
        stats_ref[...] = jnp.zeros_like(stats_ref)

    upd = jnp.concatenate(
        [jnp.sum(h, axis=0, keepdims=True),
         jnp.sum(h * h, axis=0, keepdims=True),
         jnp.zeros((6, HID), jnp.float32)], axis=0)
    stats_ref[...] += upd


def _mix(x, q, wfc, bfc, w2, b2, eps):
    return pl.pallas_call(
        _mix_body,
        grid=(GRID_R,),
        in_specs=[
            pl.BlockSpec((RB, HID), lambda i: (i, 0)),
            pl.BlockSpec((1, RB, HID), lambda i: (0, i, 0)),
            pl.BlockSpec((1, RB, HID), lambda i: (1, i, 0)),
            pl.BlockSpec((HID, HID), lambda i: (0, 0)),
            pl.BlockSpec((1, HID), lambda i: (0, 0)),
            pl.BlockSpec((HID, HID), lambda i: (0, 0)),
            pl.BlockSpec((1, HID), lambda i: (0, 0)),
            pl.BlockSpec((RB, 1), lambda i: (i, 0)),
        ],
        out_specs=[
            pl.BlockSpec((RB, HID), lambda i: (i, 0)),
            pl.BlockSpec((8, HID), lambda i: (0, 0)),
        ],
        out_shape=[
            jax.ShapeDtypeStruct((N, HID), jnp.float32),
            jax.ShapeDtypeStruct((8, HID), jnp.float32),
        ],
    )(x, q, q, wfc, bfc, w2, b2, eps)


def _gram_body(h_ref, g_ref, gt_ref, a_ref, b_ref, ret_ref, hbn_ref):
    ret_ref[...] = jnp.dot(g_ref[...], gt_ref[...],
                           preferred_element_type=jnp.float32) * 0.5

    @pl.when(pl.program_id(1) == 0)
    def _():
        hbn_ref[...] = h_ref[...] * a_ref[...] + b_ref[...]


def _gram(h, g, gt, a, b):
    return pl.pallas_call(
        _gram_body,
        grid=(GRID_GM, GRID_GN),
        in_specs=[
            pl.BlockSpec((GBM, HID), lambda i, j: (i, 0)),
            pl.BlockSpec((GBM, 2 * HID), lambda i, j: (i, 0)),
            pl.BlockSpec((2 * HID, GBN), lambda i, j: (0, j)),
            pl.BlockSpec((1, HID), lambda i, j: (0, 0)),
            pl.BlockSpec((1, HID), lambda i, j: (0, 0)),
        ],
        out_specs=[
            pl.BlockSpec((GBM, GBN), lambda i, j: (i, j)),
            pl.BlockSpec((GBM, HID), lambda i, j: (i, 0)),
        ],
        out_shape=[
            jax.ShapeDtypeStruct((N, N), jnp.float32),
            jax.ShapeDtypeStruct((N, HID), jnp.float32),
        ],
    )(h, g, gt, a, b)


def kernel(x, adj, edge_index, W_fc, b_fc, W1, b1, W2, b2, epsilon, gamma,
           beta):
    src3 = edge_index[0].reshape(NW, NCHUNK, CHUNK)
    dst3 = edge_index[1].reshape(NW, NCHUNK, CHUNK)
    zeros = jnp.zeros((RPT, HID), jnp.float32)

    p = _segsum(x, src3, dst3, zeros)
    h2 = _lin_relu(p, W1, b1.reshape(1, HID))
    q = _segsum(h2, src3, dst3, zeros)
    h, stats = _mix(x, q, W_fc, b_fc.reshape(1, HID), W2, b2.reshape(1, HID),
                    epsilon.reshape(N, 1))

    mean = stats[0] / N
    var = stats[1] / N - mean * mean
    a = gamma * lax.rsqrt(var + 1e-5)
    b = beta - mean * a
    g = jnp.concatenate([h, x], axis=1).astype(jnp.bfloat16)
    ret, h_bn = _gram(h, g, g.T, a.reshape(1, HID), b.reshape(1, HID))
    return ret, h_bn


# speedup vs baseline: 1.0207x; 1.0207x over previous
import functools

import jax
import jax.numpy as jnp
from jax import lax
from jax.experimental import pallas as pl
from jax.experimental.pallas import tpu as pltpu
from jax.experimental.pallas import tpu_sc as plsc

N = 10000
E = 320000
HID = 128

NC, NS = 2, 16
NW = NC * NS
EPW = E // NW
CHUNK = 125
NCHUNK = EPW // CHUNK
HCHUNK = NCHUNK // 2
NPAD = 10240
RPT = NPAD // NS

RB = 1000
GRID_R = N // RB
GBM = 2048
GBN = 2560
GRID_GM = pl.cdiv(N, GBM)
GRID_GN = pl.cdiv(N, GBN)


def _segsum_body(table, src3, dst3, zeros, out, src_v, dst_v, rows0_v,
                 rows1_v, agg_sh, sem0, sem1):
    c = lax.axis_index("c")
    s = lax.axis_index("s")
    wid = c * NS + s
    pltpu.sync_copy(zeros, agg_sh.at[pl.ds(s * RPT, RPT)])
    plsc.subcore_barrier()

    for half in range(2):
        pltpu.sync_copy(src3.at[wid, pl.ds(half * HCHUNK, HCHUNK)], src_v)
        pltpu.sync_copy(dst3.at[wid, pl.ds(half * HCHUNK, HCHUNK)], dst_v)
        pltpu.async_copy(table.at[src_v.at[0]], rows0_v, sem0)

        def pair(t, carry):
            j0 = 2 * t
            pltpu.make_async_copy(table.at[src_v.at[j0]], rows0_v,
                                  sem0).wait()
            pltpu.async_copy(table.at[src_v.at[j0 + 1]], rows1_v, sem1)
            pltpu.sync_copy(rows0_v, agg_sh.at[dst_v.at[j0]], add=True)
            pltpu.make_async_copy(table.at[src_v.at[j0 + 1]], rows1_v,
                                  sem1).wait()

            @pl.when(t + 1 < HCHUNK // 2)
            def _():
                pltpu.async_copy(table.at[src_v.at[j0 + 2]], rows0_v, sem0)

            pltpu.sync_copy(rows1_v, agg_sh.at[dst_v.at[j0 + 1]], add=True)
            return carry

        lax.fori_loop(0, HCHUNK // 2, pair, 0)
    plsc.subcore_barrier()
    pltpu.sync_copy(agg_sh.at[pl.ds(s * RPT, RPT)],
                    out.at[c, pl.ds(s * RPT, RPT)])


@functools.lru_cache(maxsize=1)
def _build_segsum():
    return pl.kernel(
        _segsum_body,
        out_type=jax.ShapeDtypeStruct((NC, NPAD, HID), jnp.float32),
        mesh=plsc.VectorSubcoreMesh(core_axis_name="c", subcore_axis_name="s",
                                    num_cores=NC, num_subcores=NS),
        scratch_types=[
            pltpu.VMEM((HCHUNK, CHUNK), jnp.int32),
            pltpu.VMEM((HCHUNK, CHUNK), jnp.int32),
            pltpu.VMEM((CHUNK, HID), jnp.float32),
            pltpu.VMEM((CHUNK, HID), jnp.float32),
            pltpu.VMEM_SHARED((NPAD, HID), jnp.float32),
            pltpu.SemaphoreType.DMA,
            pltpu.SemaphoreType.DMA,
        ],
    )


def _segsum(table, src3, dst3, zeros):
    return _build_segsum()(table, src3, dst3, zeros)


def _lin_relu_body(p0_ref, p1_ref, w_ref, b_ref, o_ref):
    s = p0_ref[0] + p1_ref[0]
    o_ref[...] = jnp.maximum(
        jnp.dot(s, w_ref[...], preferred_element_type=jnp.float32)
        + b_ref[...], 0.0)


def _lin_relu(p, w, b):
    return pl.pallas_call(
        _lin_relu_body,
        grid=(GRID_R,),
        in_specs=[
            pl.BlockSpec((1, RB, HID), lambda i: (0, i, 0)),
            pl.BlockSpec((1, RB, HID), lambda i: (1, i, 0)),
            pl.BlockSpec((HID, HID), lambda i: (0, 0)),
            pl.BlockSpec((1, HID), lambda i: (0, 0)),
        ],
        out_specs=pl.BlockSpec((RB, HID), lambda i: (i, 0)),
        out_shape=jax.ShapeDtypeStruct((N, HID), jnp.float32),
    )(p, p, w, b)


def _mix_body(x_ref, q0_ref, q1_ref, wfc_ref, bfc_ref, w2_ref, b2_ref,
              eps_ref, h_ref, stats_ref):
    h1 = (jnp.dot(x_ref[...], wfc_ref[...],
                  preferred_element_type=jnp.float32) + bfc_ref[...])
    agg = q0_ref[0] + q1_ref[0]
    h2 = (jnp.dot(agg, w2_ref[...],
                  preferred_element_type=jnp.float32) + b2_ref[...])
    e = eps_ref[...]
    h = h1 + e * (h2 - h1)
    h_ref[...] = h

    @pl.when(pl.program_id(0) == 0)
    def _():
        stats_ref[...] = jnp.zeros_like(stats_ref)

    upd = jnp.concatenate(
        [jnp.sum(h, axis=0, keepdims=True),
         jnp.sum(h * h, axis=0, keepdims=True),
         jnp.zeros((6, HID), jnp.float32)], axis=0)
    stats_ref[...] += upd


def _mix(x, q, wfc, bfc, w2, b2, eps):
    return pl.pallas_call(
        _mix_body,
        grid=(GRID_R,),
        in_specs=[
            pl.BlockSpec((RB, HID), lambda i: (i, 0)),
            pl.BlockSpec((1, RB, HID), lambda i: (0, i, 0)),
            pl.BlockSpec((1, RB, HID), lambda i: (1, i, 0)),
            pl.BlockSpec((HID, HID), lambda i: (0, 0)),
            pl.BlockSpec((1, HID), lambda i: (0, 0)),
            pl.BlockSpec((HID, HID), lambda i: (0, 0)),
            pl.BlockSpec((1, HID), lambda i: (0, 0)),
            pl.BlockSpec((RB, 1), lambda i: (i, 0)),
        ],
        out_specs=[
            pl.BlockSpec((RB, HID), lambda i: (i, 0)),
            pl.BlockSpec((8, HID), lambda i: (0, 0)),
        ],
        out_shape=[
            jax.ShapeDtypeStruct((N, HID), jnp.float32),
            jax.ShapeDtypeStruct((8, HID), jnp.float32),
        ],
    )(x, q, q, wfc, bfc, w2, b2, eps)


def _gram_body(h_ref, g_ref, gt_ref, a_ref, b_ref, ret_ref, hbn_ref):
    ret_ref[...] = jnp.dot(g_ref[...], gt_ref[...],
                           preferred_element_type=jnp.float32) * 0.5

    @pl.when(pl.program_id(1) == 0)
    def _():
        hbn_ref[...] = h_ref[...] * a_ref[...] + b_ref[...]


def _gram(h, g, gt, a, b):
    return pl.pallas_call(
        _gram_body,
        grid=(GRID_GM, GRID_GN),
        in_specs=[
            pl.BlockSpec((GBM, HID), lambda i, j: (i, 0)),
            pl.BlockSpec((GBM, 2 * HID), lambda i, j: (i, 0)),
            pl.BlockSpec((2 * HID, GBN), lambda i, j: (0, j)),
            pl.BlockSpec((1, HID), lambda i, j: (0, 0)),
            pl.BlockSpec((1, HID), lambda i, j: (0, 0)),
        ],
        out_specs=[
            pl.BlockSpec((GBM, GBN), lambda i, j: (i, j)),
            pl.BlockSpec((GBM, HID), lambda i, j: (i, 0)),
        ],
        out_shape=[
            jax.ShapeDtypeStruct((N, N), jnp.float32),
            jax.ShapeDtypeStruct((N, HID), jnp.float32),
        ],
    )(h, g, gt, a, b)


def kernel(x, adj, edge_index, W_fc, b_fc, W1, b1, W2, b2, epsilon, gamma,
           beta):
    src3 = edge_index[0].reshape(NW, NCHUNK, CHUNK)
    dst3 = edge_index[1].reshape(NW, NCHUNK, CHUNK)
    zeros = jnp.zeros((RPT, HID), jnp.float32)

    p = _segsum(x, src3, dst3, zeros)
    h2 = _lin_relu(p, W1, b1.reshape(1, HID))
    q = _segsum(h2, src3, dst3, zeros)
    h, stats = _mix(x, q, W_fc, b_fc.reshape(1, HID), W2, b2.reshape(1, HID),
                    epsilon.reshape(N, 1))

    mean = stats[0] / N
    var = stats[1] / N - mean * mean
    a = gamma * lax.rsqrt(var + 1e-5)
    b = beta - mean * a
    g = jnp.concatenate([h, x], axis=1).astype(jnp.bfloat16)
    ret, h_bn = _gram(h, g, g.T, a.reshape(1, HID), b.reshape(1, HID))
    return ret, h_bn


# speedup vs baseline: 1.0331x; 1.0121x over previous
import functools

import jax
import jax.numpy as jnp
from jax import lax
from jax.experimental import pallas as pl
from jax.experimental.pallas import tpu as pltpu
from jax.experimental.pallas import tpu_sc as plsc

N = 10000
E = 320000
HID = 128

NC, NS = 2, 16
NW = NC * NS
EPW = E // NW
CHUNK = 125
NCHUNK = EPW // CHUNK
HCHUNK = NCHUNK // 2
NPAD = 10240
RPT = NPAD // NS

RB = 2000
GRID_R = N // RB
GBM = 2048
GBN = 2048
GRID_GM = pl.cdiv(N, GBM)
GRID_GN = pl.cdiv(N, GBN)


def _segsum_body(table, src3, dst3, zeros, out, src_v, dst_v, rows0_v,
                 rows1_v, agg_sh, sem0, sem1):
    c = lax.axis_index("c")
    s = lax.axis_index("s")
    wid = c * NS + s
    pltpu.sync_copy(zeros, agg_sh.at[pl.ds(s * RPT, RPT)])
    plsc.subcore_barrier()

    for half in range(2):
        pltpu.sync_copy(src3.at[wid, pl.ds(half * HCHUNK, HCHUNK)], src_v)
        pltpu.sync_copy(dst3.at[wid, pl.ds(half * HCHUNK, HCHUNK)], dst_v)
        pltpu.async_copy(table.at[src_v.at[0]], rows0_v, sem0)

        def pair(t, carry):
            j0 = 2 * t
            pltpu.make_async_copy(table.at[src_v.at[j0]], rows0_v,
                                  sem0).wait()
            pltpu.async_copy(table.at[src_v.at[j0 + 1]], rows1_v, sem1)
            pltpu.sync_copy(rows0_v, agg_sh.at[dst_v.at[j0]], add=True)
            pltpu.make_async_copy(table.at[src_v.at[j0 + 1]], rows1_v,
                                  sem1).wait()

            @pl.when(t + 1 < HCHUNK // 2)
            def _():
                pltpu.async_copy(table.at[src_v.at[j0 + 2]], rows0_v, sem0)

            pltpu.sync_copy(rows1_v, agg_sh.at[dst_v.at[j0 + 1]], add=True)
            return carry

        lax.fori_loop(0, HCHUNK // 2, pair, 0)
    plsc.subcore_barrier()
    pltpu.sync_copy(agg_sh.at[pl.ds(s * RPT, RPT)],
                    out.at[c, pl.ds(s * RPT, RPT)])


@functools.lru_cache(maxsize=1)
def _build_segsum():
    return pl.kernel(
        _segsum_body,
        out_type=jax.ShapeDtypeStruct((NC, NPAD, HID), jnp.float32),
        mesh=plsc.VectorSubcoreMesh(core_axis_name="c", subcore_axis_name="s",
                                    num_cores=NC, num_subcores=NS),
        scratch_types=[
            pltpu.VMEM((HCHUNK, CHUNK), jnp.int32),
            pltpu.VMEM((HCHUNK, CHUNK), jnp.int32),
            pltpu.VMEM((CHUNK, HID), jnp.float32),
            pltpu.VMEM((CHUNK, HID), jnp.float32),
            pltpu.VMEM_SHARED((NPAD, HID), jnp.float32),
            pltpu.SemaphoreType.DMA,
            pltpu.SemaphoreType.DMA,
        ],
    )


def _segsum(table, src3, dst3, zeros):
    return _build_segsum()(table, src3, dst3, zeros)


def _lin_relu_body(p0_ref, p1_ref, w_ref, b_ref, o_ref):
    s = p0_ref[0] + p1_ref[0]
    o_ref[...] = jnp.maximum(
        jnp.dot(s, w_ref[...], preferred_element_type=jnp.float32)
        + b_ref[...], 0.0)


def _lin_relu(p, w, b):
    return pl.pallas_call(
        _lin_relu_body,
        grid=(GRID_R,),
        in_specs=[
            pl.BlockSpec((1, RB, HID), lambda i: (0, i, 0)),
            pl.BlockSpec((1, RB, HID), lambda i: (1, i, 0)),
            pl.BlockSpec((HID, HID), lambda i: (0, 0)),
            pl.BlockSpec((1, HID), lambda i: (0, 0)),
        ],
        out_specs=pl.BlockSpec((RB, HID), lambda i: (i, 0)),
        out_shape=jax.ShapeDtypeStruct((N, HID), jnp.float32),
    )(p, p, w, b)


def _mix_body(x_ref, q0_ref, q1_ref, wfc_ref, bfc_ref, w2_ref, b2_ref,
              eps_ref, h_ref, stats_ref):
    h1 = (jnp.dot(x_ref[...], wfc_ref[...],
                  preferred_element_type=jnp.float32) + bfc_ref[...])
    agg = q0_ref[0] + q1_ref[0]
    h2 = (jnp.dot(agg, w2_ref[...],
                  preferred_element_type=jnp.float32) + b2_ref[...])
    e = eps_ref[...]
    h = h1 + e * (h2 - h1)
    h_ref[...] = h

    @pl.when(pl.program_id(0) == 0)
    def _():
        stats_ref[...] = jnp.zeros_like(stats_ref)

    upd = jnp.concatenate(
        [jnp.sum(h, axis=0, keepdims=True),
         jnp.sum(h * h, axis=0, keepdims=True),
         jnp.zeros((6, HID), jnp.float32)], axis=0)
    stats_ref[...] += upd


def _mix(x, q, wfc, bfc, w2, b2, eps):
    return pl.pallas_call(
        _mix_body,
        grid=(GRID_R,),
        in_specs=[
            pl.BlockSpec((RB, HID), lambda i: (i, 0)),
            pl.BlockSpec((1, RB, HID), lambda i: (0, i, 0)),
            pl.BlockSpec((1, RB, HID), lambda i: (1, i, 0)),
            pl.BlockSpec((HID, HID), lambda i: (0, 0)),
            pl.BlockSpec((1, HID), lambda i: (0, 0)),
            pl.BlockSpec((HID, HID), lambda i: (0, 0)),
            pl.BlockSpec((1, HID), lambda i: (0, 0)),
            pl.BlockSpec((RB, 1), lambda i: (i, 0)),
        ],
        out_specs=[
            pl.BlockSpec((RB, HID), lambda i: (i, 0)),
            pl.BlockSpec((8, HID), lambda i: (0, 0)),
        ],
        out_shape=[
            jax.ShapeDtypeStruct((N, HID), jnp.float32),
            jax.ShapeDtypeStruct((8, HID), jnp.float32),
        ],
    )(x, q, q, wfc, bfc, w2, b2, eps)


def _gram_body(h_ref, g_ref, gt_ref, a_ref, b_ref, ret_ref, hbn_ref):
    ret_ref[...] = jnp.dot(g_ref[...], gt_ref[...],
                           preferred_element_type=jnp.float32) * 0.5

    @pl.when(pl.program_id(1) == 0)
    def _():
        hbn_ref[...] = h_ref[...] * a_ref[...] + b_ref[...]


def _gram(h, g, gt, a, b):
    return pl.pallas_call(
        _gram_body,
        grid=(GRID_GM, GRID_GN),
        in_specs=[
            pl.BlockSpec((GBM, HID), lambda i, j: (i, 0)),
            pl.BlockSpec((GBM, 2 * HID), lambda i, j: (i, 0)),
            pl.BlockSpec((2 * HID, GBN), lambda i, j: (0, j)),
            pl.BlockSpec((1, HID), lambda i, j: (0, 0)),
            pl.BlockSpec((1, HID), lambda i, j: (0, 0)),
        ],
        out_specs=[
            pl.BlockSpec((GBM, GBN), lambda i, j: (i, j)),
            pl.BlockSpec((GBM, HID), lambda i, j: (i, 0)),
        ],
        out_shape=[
            jax.ShapeDtypeStruct((N, N), jnp.float32),
            jax.ShapeDtypeStruct((N, HID), jnp.float32),
        ],
    )(h, g, gt, a, b)


def kernel(x, adj, edge_index, W_fc, b_fc, W1, b1, W2, b2, epsilon, gamma,
           beta):
    src3 = edge_index[0].reshape(NW, NCHUNK, CHUNK)
    dst3 = edge_index[1].reshape(NW, NCHUNK, CHUNK)
    zeros = jnp.zeros((RPT, HID), jnp.float32)

    p = _segsum(x, src3, dst3, zeros)
    h2 = _lin_relu(p, W1, b1.reshape(1, HID))
    q = _segsum(h2, src3, dst3, zeros)
    h, stats = _mix(x, q, W_fc, b_fc.reshape(1, HID), W2, b2.reshape(1, HID),
                    epsilon.reshape(N, 1))

    mean = stats[0] / N
    var = stats[1] / N - mean * mean
    a = gamma * lax.rsqrt(var + 1e-5)
    b = beta - mean * a
    g = jnp.concatenate([h, x], axis=1).astype(jnp.bfloat16)
    ret, h_bn = _gram(h, g, g.T, a.reshape(1, HID), b.reshape(1, HID))
    return ret, h_bn


# speedup vs baseline: 1.0419x; 1.0085x over previous
import functools

import jax
import jax.numpy as jnp
from jax import lax
from jax.experimental import pallas as pl
from jax.experimental.pallas import tpu as pltpu
from jax.experimental.pallas import tpu_sc as plsc

N = 10000
E = 320000
HID = 128

NC, NS = 2, 16
NW = NC * NS
EPW = E // NW
CHUNK = 125
NCHUNK = EPW // CHUNK
HCHUNK = NCHUNK // 2
NPAD = 10240
RPT = NPAD // NS

RB = 2000
GRID_R = N // RB
GBM = 2048
GBN = 2048
GRID_GM = pl.cdiv(N, GBM)
GRID_GN = pl.cdiv(N, GBN)


def _segsum_body(table, src3, dst3, zeros, out, src_v, dst_v, rows0_v,
                 rows1_v, agg_sh, sem0, sem1):
    c = lax.axis_index("c")
    s = lax.axis_index("s")
    wid = c * NS + s
    pltpu.sync_copy(zeros, agg_sh.at[pl.ds(s * RPT, RPT)])
    plsc.subcore_barrier()

    for half in range(2):
        pltpu.sync_copy(src3.at[wid, pl.ds(half * HCHUNK, HCHUNK)], src_v)
        pltpu.sync_copy(dst3.at[wid, pl.ds(half * HCHUNK, HCHUNK)], dst_v)
        pltpu.async_copy(table.at[src_v.at[0]], rows0_v, sem0)

        def pair(t, carry):
            j0 = 2 * t
            pltpu.make_async_copy(table.at[src_v.at[j0]], rows0_v,
                                  sem0).wait()
            pltpu.async_copy(table.at[src_v.at[j0 + 1]], rows1_v, sem1)
            pltpu.sync_copy(rows0_v, agg_sh.at[dst_v.at[j0]], add=True)
            pltpu.make_async_copy(table.at[src_v.at[j0 + 1]], rows1_v,
                                  sem1).wait()

            @pl.when(t + 1 < HCHUNK // 2)
            def _():
                pltpu.async_copy(table.at[src_v.at[j0 + 2]], rows0_v, sem0)

            pltpu.sync_copy(rows1_v, agg_sh.at[dst_v.at[j0 + 1]], add=True)
            return carry

        lax.fori_loop(0, HCHUNK // 2, pair, 0)
    plsc.subcore_barrier()
    pltpu.sync_copy(agg_sh.at[pl.ds(s * RPT, RPT)],
                    out.at[c, pl.ds(s * RPT, RPT)])


@functools.lru_cache(maxsize=1)
def _build_segsum():
    return pl.kernel(
        _segsum_body,
        out_type=jax.ShapeDtypeStruct((NC, NPAD, HID), jnp.float32),
        mesh=plsc.VectorSubcoreMesh(core_axis_name="c", subcore_axis_name="s",
                                    num_cores=NC, num_subcores=NS),
        scratch_types=[
            pltpu.VMEM((HCHUNK, CHUNK), jnp.int32),
            pltpu.VMEM((HCHUNK, CHUNK), jnp.int32),
            pltpu.VMEM((CHUNK, HID), jnp.float32),
            pltpu.VMEM((CHUNK, HID), jnp.float32),
            pltpu.VMEM_SHARED((NPAD, HID), jnp.float32),
            pltpu.SemaphoreType.DMA,
            pltpu.SemaphoreType.DMA,
        ],
    )


def _segsum(table, src3, dst3, zeros):
    return _build_segsum()(table, src3, dst3, zeros)


def _lin_relu_body(p0_ref, p1_ref, w_ref, b_ref, o_ref):
    s = p0_ref[0] + p1_ref[0]
    o_ref[...] = jnp.maximum(
        jnp.dot(s, w_ref[...], preferred_element_type=jnp.float32)
        + b_ref[...], 0.0)


def _lin_relu(p, w, b):
    return pl.pallas_call(
        _lin_relu_body,
        grid=(GRID_R,),
        in_specs=[
            pl.BlockSpec((1, RB, HID), lambda i: (0, i, 0)),
            pl.BlockSpec((1, RB, HID), lambda i: (1, i, 0)),
            pl.BlockSpec((HID, HID), lambda i: (0, 0)),
            pl.BlockSpec((1, HID), lambda i: (0, 0)),
        ],
        out_specs=pl.BlockSpec((RB, HID), lambda i: (i, 0)),
        out_shape=jax.ShapeDtypeStruct((N, HID), jnp.float32),
    )(p, p, w, b)


def _mix_body(x_ref, q0_ref, q1_ref, wfc_ref, bfc_ref, w2_ref, b2_ref,
              eps_ref, h_ref, g_ref, stats_ref):
    x = x_ref[...]
    h1 = (jnp.dot(x, wfc_ref[...],
                  preferred_element_type=jnp.float32) + bfc_ref[...])
    agg = q0_ref[0] + q1_ref[0]
    h2 = (jnp.dot(agg, w2_ref[...],
                  preferred_element_type=jnp.float32) + b2_ref[...])
    e = eps_ref[...]
    h = h1 + e * (h2 - h1)
    h_ref[...] = h
    g_ref[...] = jnp.concatenate([h, x], axis=1).astype(jnp.bfloat16)

    @pl.when(pl.program_id(0) == 0)
    def _():
        stats_ref[...] = jnp.zeros_like(stats_ref)

    upd = jnp.concatenate(
        [jnp.sum(h, axis=0, keepdims=True),
         jnp.sum(h * h, axis=0, keepdims=True),
         jnp.zeros((6, HID), jnp.float32)], axis=0)
    stats_ref[...] += upd


def _mix(x, q, wfc, bfc, w2, b2, eps):
    return pl.pallas_call(
        _mix_body,
        grid=(GRID_R,),
        in_specs=[
            pl.BlockSpec((RB, HID), lambda i: (i, 0)),
            pl.BlockSpec((1, RB, HID), lambda i: (0, i, 0)),
            pl.BlockSpec((1, RB, HID), lambda i: (1, i, 0)),
            pl.BlockSpec((HID, HID), lambda i: (0, 0)),
            pl.BlockSpec((1, HID), lambda i: (0, 0)),
            pl.BlockSpec((HID, HID), lambda i: (0, 0)),
            pl.BlockSpec((1, HID), lambda i: (0, 0)),
            pl.BlockSpec((RB, 1), lambda i: (i, 0)),
        ],
        out_specs=[
            pl.BlockSpec((RB, HID), lambda i: (i, 0)),
            pl.BlockSpec((RB, 2 * HID), lambda i: (i, 0)),
            pl.BlockSpec((8, HID), lambda i: (0, 0)),
        ],
        out_shape=[
            jax.ShapeDtypeStruct((N, HID), jnp.float32),
            jax.ShapeDtypeStruct((N, 2 * HID), jnp.bfloat16),
            jax.ShapeDtypeStruct((8, HID), jnp.float32),
        ],
    )(x, q, q, wfc, bfc, w2, b2, eps)


def _gram_body(h_ref, g_ref, gt_ref, a_ref, b_ref, ret_ref, hbn_ref):
    ret_ref[...] = jnp.dot(g_ref[...], gt_ref[...],
                           preferred_element_type=jnp.float32) * 0.5

    @pl.when(pl.program_id(1) == 0)
    def _():
        hbn_ref[...] = h_ref[...] * a_ref[...] + b_ref[...]


def _gram(h, g, gt, a, b):
    return pl.pallas_call(
        _gram_body,
        grid=(GRID_GM, GRID_GN),
        in_specs=[
            pl.BlockSpec((GBM, HID), lambda i, j: (i, 0)),
            pl.BlockSpec((GBM, 2 * HID), lambda i, j: (i, 0)),
            pl.BlockSpec((2 * HID, GBN), lambda i, j: (0, j)),
            pl.BlockSpec((1, HID), lambda i, j: (0, 0)),
            pl.BlockSpec((1, HID), lambda i, j: (0, 0)),
        ],
        out_specs=[
            pl.BlockSpec((GBM, GBN), lambda i, j: (i, j)),
            pl.BlockSpec((GBM, HID), lambda i, j: (i, 0)),
        ],
        out_shape=[
            jax.ShapeDtypeStruct((N, N), jnp.float32),
            jax.ShapeDtypeStruct((N, HID), jnp.float32),
        ],
    )(h, g, gt, a, b)


def kernel(x, adj, edge_index, W_fc, b_fc, W1, b1, W2, b2, epsilon, gamma,
           beta):
    src3 = edge_index[0].reshape(NW, NCHUNK, CHUNK)
    dst3 = edge_index[1].reshape(NW, NCHUNK, CHUNK)
    zeros = jnp.zeros((RPT, HID), jnp.float32)

    p = _segsum(x, src3, dst3, zeros)
    h2 = _lin_relu(p, W1, b1.reshape(1, HID))
    q = _segsum(h2, src3, dst3, zeros)
    h, g, stats = _mix(x, q, W_fc, b_fc.reshape(1, HID), W2,
                       b2.reshape(1, HID), epsilon.reshape(N, 1))

    mean = stats[0] / N
    var = stats[1] / N - mean * mean
    a = gamma * lax.rsqrt(var + 1e-5)
    b = beta - mean * a
    ret, h_bn = _gram(h, g, g.T, a.reshape(1, HID), b.reshape(1, HID))
    return ret, h_bn


# speedup vs baseline: 1.0430x; 1.0011x over previous
import functools

import jax
import jax.numpy as jnp
from jax import lax
from jax.experimental import pallas as pl
from jax.experimental.pallas import tpu as pltpu
from jax.experimental.pallas import tpu_sc as plsc

N = 10000
E = 320000
HID = 128

NC, NS = 2, 16
NW = NC * NS
EPW = E // NW
CHUNK = 125
NCHUNK = EPW // CHUNK
HCHUNK = NCHUNK // 2
NPAD = 10240
RPT = NPAD // NS

RB = 2000
GRID_R = N // RB
GBM = 2000
GBN = 2048
GRID_GM = pl.cdiv(N, GBM)
GRID_GN = pl.cdiv(N, GBN)


def _segsum_body(table, src3, dst3, zeros, out, src_v, dst_v, rows0_v,
                 rows1_v, agg_sh, sem0, sem1):
    c = lax.axis_index("c")
    s = lax.axis_index("s")
    wid = c * NS + s
    pltpu.sync_copy(zeros, agg_sh.at[pl.ds(s * RPT, RPT)])
    plsc.subcore_barrier()

    for half in range(2):
        pltpu.sync_copy(src3.at[wid, pl.ds(half * HCHUNK, HCHUNK)], src_v)
        pltpu.sync_copy(dst3.at[wid, pl.ds(half * HCHUNK, HCHUNK)], dst_v)
        pltpu.async_copy(table.at[src_v.at[0]], rows0_v, sem0)

        def pair(t, carry):
            j0 = 2 * t
            pltpu.make_async_copy(table.at[src_v.at[j0]], rows0_v,
                                  sem0).wait()
            pltpu.async_copy(table.at[src_v.at[j0 + 1]], rows1_v, sem1)
            pltpu.sync_copy(rows0_v, agg_sh.at[dst_v.at[j0]], add=True)
            pltpu.make_async_copy(table.at[src_v.at[j0 + 1]], rows1_v,
                                  sem1).wait()

            @pl.when(t + 1 < HCHUNK // 2)
            def _():
                pltpu.async_copy(table.at[src_v.at[j0 + 2]], rows0_v, sem0)

            pltpu.sync_copy(rows1_v, agg_sh.at[dst_v.at[j0 + 1]], add=True)
            return carry

        lax.fori_loop(0, HCHUNK // 2, pair, 0)
    plsc.subcore_barrier()
    pltpu.sync_copy(agg_sh.at[pl.ds(s * RPT, RPT)],
                    out.at[c, pl.ds(s * RPT, RPT)])


@functools.lru_cache(maxsize=1)
def _build_segsum():
    return pl.kernel(
        _segsum_body,
        out_type=jax.ShapeDtypeStruct((NC, NPAD, HID), jnp.float32),
        mesh=plsc.VectorSubcoreMesh(core_axis_name="c", subcore_axis_name="s",
                                    num_cores=NC, num_subcores=NS),
        scratch_types=[
            pltpu.VMEM((HCHUNK, CHUNK), jnp.int32),
            pltpu.VMEM((HCHUNK, CHUNK), jnp.int32),
            pltpu.VMEM((CHUNK, HID), jnp.float32),
            pltpu.VMEM((CHUNK, HID), jnp.float32),
            pltpu.VMEM_SHARED((NPAD, HID), jnp.float32),
            pltpu.SemaphoreType.DMA,
            pltpu.SemaphoreType.DMA,
        ],
    )


def _segsum(table, src3, dst3, zeros):
    return _build_segsum()(table, src3, dst3, zeros)


def _lin_relu_body(p0_ref, p1_ref, w_ref, b_ref, o_ref):
    s = p0_ref[0] + p1_ref[0]
    o_ref[...] = jnp.maximum(
        jnp.dot(s, w_ref[...], preferred_element_type=jnp.float32)
        + b_ref[...], 0.0)


def _lin_relu(p, w, b):
    return pl.pallas_call(
        _lin_relu_body,
        grid=(GRID_R,),
        in_specs=[
            pl.BlockSpec((1, RB, HID), lambda i: (0, i, 0)),
            pl.BlockSpec((1, RB, HID), lambda i: (1, i, 0)),
            pl.BlockSpec((HID, HID), lambda i: (0, 0)),
            pl.BlockSpec((1, HID), lambda i: (0, 0)),
        ],
        out_specs=pl.BlockSpec((RB, HID), lambda i: (i, 0)),
        out_shape=jax.ShapeDtypeStruct((N, HID), jnp.float32),
    )(p, p, w, b)


def _mix_body(x_ref, q0_ref, q1_ref, wfc_ref, bfc_ref, w2_ref, b2_ref,
              eps_ref, h_ref, g_ref, stats_ref):
    x = x_ref[...]
    h1 = (jnp.dot(x, wfc_ref[...],
                  preferred_element_type=jnp.float32) + bfc_ref[...])
    agg = q0_ref[0] + q1_ref[0]
    h2 = (jnp.dot(agg, w2_ref[...],
                  preferred_element_type=jnp.float32) + b2_ref[...])
    e = eps_ref[...]
    h = h1 + e * (h2 - h1)
    h_ref[...] = h
    g_ref[...] = jnp.concatenate([h, x], axis=1).astype(jnp.bfloat16)

    @pl.when(pl.program_id(0) == 0)
    def _():
        stats_ref[...] = jnp.zeros_like(stats_ref)

    upd = jnp.concatenate(
        [jnp.sum(h, axis=0, keepdims=True),
         jnp.sum(h * h, axis=0, keepdims=True),
         jnp.zeros((6, HID), jnp.float32)], axis=0)
    stats_ref[...] += upd


def _mix(x, q, wfc, bfc, w2, b2, eps):
    return pl.pallas_call(
        _mix_body,
        grid=(GRID_R,),
        in_specs=[
            pl.BlockSpec((RB, HID), lambda i: (i, 0)),
            pl.BlockSpec((1, RB, HID), lambda i: (0, i, 0)),
            pl.BlockSpec((1, RB, HID), lambda i: (1, i, 0)),
            pl.BlockSpec((HID, HID), lambda i: (0, 0)),
            pl.BlockSpec((1, HID), lambda i: (0, 0)),
            pl.BlockSpec((HID, HID), lambda i: (0, 0)),
            pl.BlockSpec((1, HID), lambda i: (0, 0)),
            pl.BlockSpec((RB, 1), lambda i: (i, 0)),
        ],
        out_specs=[
            pl.BlockSpec((RB, HID), lambda i: (i, 0)),
            pl.BlockSpec((RB, 2 * HID), lambda i: (i, 0)),
            pl.BlockSpec((8, HID), lambda i: (0, 0)),
        ],
        out_shape=[
            jax.ShapeDtypeStruct((N, HID), jnp.float32),
            jax.ShapeDtypeStruct((N, 2 * HID), jnp.bfloat16),
            jax.ShapeDtypeStruct((8, HID), jnp.float32),
        ],
    )(x, q, q, wfc, bfc, w2, b2, eps)


def _gram_body(h_ref, g_ref, gt_ref, a_ref, b_ref, ret_ref, hbn_ref):
    ret_ref[...] = jnp.dot(g_ref[...], gt_ref[...],
                           preferred_element_type=jnp.float32) * 0.5

    @pl.when(pl.program_id(1) == 0)
    def _():
        hbn_ref[...] = h_ref[...] * a_ref[...] + b_ref[...]


def _gram(h, g, gt, a, b):
    return pl.pallas_call(
        _gram_body,
        grid=(GRID_GM, GRID_GN),
        in_specs=[
            pl.BlockSpec((GBM, HID), lambda i, j: (i, 0)),
            pl.BlockSpec((GBM, 2 * HID), lambda i, j: (i, 0)),
            pl.BlockSpec((2 * HID, GBN), lambda i, j: (0, j)),
            pl.BlockSpec((1, HID), lambda i, j: (0, 0)),
            pl.BlockSpec((1, HID), lambda i, j: (0, 0)),
        ],
        out_specs=[
            pl.BlockSpec((GBM, GBN), lambda i, j: (i, j)),
            pl.BlockSpec((GBM, HID), lambda i, j: (i, 0)),
        ],
        out_shape=[
            jax.ShapeDtypeStruct((N, N), jnp.float32),
            jax.ShapeDtypeStruct((N, HID), jnp.float32),
        ],
    )(h, g, gt, a, b)


def kernel(x, adj, edge_index, W_fc, b_fc, W1, b1, W2, b2, epsilon, gamma,
           beta):
    src3 = edge_index[0].reshape(NW, NCHUNK, CHUNK)
    dst3 = edge_index[1].reshape(NW, NCHUNK, CHUNK)
    zeros = jnp.zeros((RPT, HID), jnp.float32)

    p = _segsum(x, src3, dst3, zeros)
    h2 = _lin_relu(p, W1, b1.reshape(1, HID))
    q = _segsum(h2, src3, dst3, zeros)
    h, g, stats = _mix(x, q, W_fc, b_fc.reshape(1, HID), W2,
                       b2.reshape(1, HID), epsilon.reshape(N, 1))

    mean = stats[0] / N
    var = stats[1] / N - mean * mean
    a = gamma * lax.rsqrt(var + 1e-5)
    b = beta - mean * a
    ret, h_bn = _gram(h, g, g.T, a.reshape(1, HID), b.reshape(1, HID))
    return ret, h_bn
